# baseline (speedup 1.0000x reference)
import jax
import jax.numpy as jnp
from jax.experimental import pallas as pl
from jax.experimental.pallas import tpu as pltpu

_MASK_VALUE = -10000000.0


def _pooler_kernel(t_ref, k_ref, u_ref, mb_ref, wd_ref, bd_ref, wm_ref,
                   bm_ref, out_ref, attn_ref):
    BU = t_ref.shape[0]
    C = t_ref.shape[2]
    E = u_ref.shape[2]
    w1 = wd_ref[:, :C]
    w2 = wd_ref[:, C:]
    b = bd_ref[0, 0]
    for u in range(BU):
        t = t_ref[u]
        k = k_ref[u]
        s_t = jax.lax.dot_general(t, w1, (((1,), (1,)), ((), ())),
                                  preferred_element_type=jnp.float32)
        s_kb = jax.lax.dot_general(w2, k, (((1,), (1,)), ((), ())),
                                   preferred_element_type=jnp.float32) + b
        e = jnp.exp(jnp.tanh(s_t + s_kb)) * mb_ref[u]
        ones_j = jnp.ones((e.shape[1], 1), jnp.float32)
        s = jnp.dot(e, ones_j, preferred_element_type=jnp.float32)
        pooled_u = jnp.dot(e, k, preferred_element_type=jnp.float32)
        r = 1.0 / s
        attn_ref[u] = e * r
        u_part = jnp.dot(u_ref[u], wm_ref[:E, :],
                         preferred_element_type=jnp.float32)
        h = jnp.dot(pooled_u * r, wm_ref[E:, :],
                    preferred_element_type=jnp.float32)
        out_ref[u] = jnp.maximum(h + u_part + bm_ref[:], 0.0)


def kernel(target_items_context, interacted_items_context, user_embeds,
           attention_mask, w_dense, b_dense, W_mlp, b_mlp):
    U, I, C = target_items_context.shape
    J = interacted_items_context.shape[1]
    E = user_embeds.shape[1]
    BU = 16
    mask01 = attention_mask.astype(jnp.float32).reshape(U, 1, J)
    users3 = user_embeds.reshape(U, 1, E)
    wd = w_dense.reshape(1, 2 * C)
    bd = b_dense.reshape(1, 1)
    bm = b_mlp.reshape(1, C)
    out, attn = pl.pallas_call(
        _pooler_kernel,
        grid=(U // BU,),
        in_specs=[
            pl.BlockSpec((BU, I, C), lambda u: (u, 0, 0)),
            pl.BlockSpec((BU, J, C), lambda u: (u, 0, 0)),
            pl.BlockSpec((BU, 1, E), lambda u: (u, 0, 0)),
            pl.BlockSpec((BU, 1, J), lambda u: (u, 0, 0)),
            pl.BlockSpec((1, 2 * C), lambda u: (0, 0)),
            pl.BlockSpec((1, 1), lambda u: (0, 0)),
            pl.BlockSpec((E + C, C), lambda u: (0, 0)),
            pl.BlockSpec((1, C), lambda u: (0, 0)),
        ],
        out_specs=[
            pl.BlockSpec((BU, I, C), lambda u: (u, 0, 0)),
            pl.BlockSpec((BU, I, J), lambda u: (u, 0, 0)),
        ],
        out_shape=[
            jax.ShapeDtypeStruct((U, I, C), jnp.float32),
            jax.ShapeDtypeStruct((U, I, J), jnp.float32),
        ],
        compiler_params=pltpu.CompilerParams(
            dimension_semantics=("parallel",),
            vmem_limit_bytes=56 * 1024 * 1024,
        ),
    )(target_items_context, interacted_items_context, users3,
      mask01, wd, bd, W_mlp, bm)
    return out, attn


# speedup vs baseline: 1.0185x; 1.0185x over previous
import jax
import jax.numpy as jnp
from jax.experimental import pallas as pl
from jax.experimental.pallas import tpu as pltpu

_MASK_VALUE = -10000000.0


def _pooler_kernel(t_ref, k_ref, u_ref, mb_ref, wd_ref, bd_ref, wm_ref,
                   bm_ref, out_ref, attn_ref):
    BU = t_ref.shape[0]
    C = t_ref.shape[2]
    E = u_ref.shape[2]
    w1 = wd_ref[:, :C]
    w2 = wd_ref[:, C:]
    b = bd_ref[0, 0]
    for u in range(BU):
        t = t_ref[u]
        k = k_ref[u]
        s_t = jax.lax.dot_general(t, w1, (((1,), (1,)), ((), ())),
                                  preferred_element_type=jnp.float32)
        s_kb = jax.lax.dot_general(w2, k, (((1,), (1,)), ((), ())),
                                   preferred_element_type=jnp.float32) + b
        e = jnp.exp(jnp.tanh(s_t + s_kb)) * mb_ref[u]
        ones_j = jnp.ones((e.shape[1], 1), jnp.float32)
        s = jnp.dot(e, ones_j, preferred_element_type=jnp.float32)
        pooled_u = jnp.dot(e, k, preferred_element_type=jnp.float32)
        r = 1.0 / s
        attn_ref[u] = e * r
        u_part = jnp.dot(u_ref[u], wm_ref[:E, :],
                         preferred_element_type=jnp.float32)
        h = jnp.dot(pooled_u * r, wm_ref[E:, :],
                    preferred_element_type=jnp.float32)
        out_ref[u] = jnp.maximum(h + u_part + bm_ref[:], 0.0)


def kernel(target_items_context, interacted_items_context, user_embeds,
           attention_mask, w_dense, b_dense, W_mlp, b_mlp):
    U, I, C = target_items_context.shape
    J = interacted_items_context.shape[1]
    E = user_embeds.shape[1]
    BU = 8
    mask01 = attention_mask.astype(jnp.float32).reshape(U, 1, J)
    users3 = user_embeds.reshape(U, 1, E)
    wd = w_dense.reshape(1, 2 * C)
    bd = b_dense.reshape(1, 1)
    bm = b_mlp.reshape(1, C)
    out, attn = pl.pallas_call(
        _pooler_kernel,
        grid=(U // BU,),
        in_specs=[
            pl.BlockSpec((BU, I, C), lambda u: (u, 0, 0)),
            pl.BlockSpec((BU, J, C), lambda u: (u, 0, 0)),
            pl.BlockSpec((BU, 1, E), lambda u: (u, 0, 0)),
            pl.BlockSpec((BU, 1, J), lambda u: (u, 0, 0)),
            pl.BlockSpec((1, 2 * C), lambda u: (0, 0)),
            pl.BlockSpec((1, 1), lambda u: (0, 0)),
            pl.BlockSpec((E + C, C), lambda u: (0, 0)),
            pl.BlockSpec((1, C), lambda u: (0, 0)),
        ],
        out_specs=[
            pl.BlockSpec((BU, I, C), lambda u: (u, 0, 0)),
            pl.BlockSpec((BU, I, J), lambda u: (u, 0, 0)),
        ],
        out_shape=[
            jax.ShapeDtypeStruct((U, I, C), jnp.float32),
            jax.ShapeDtypeStruct((U, I, J), jnp.float32),
        ],
        compiler_params=pltpu.CompilerParams(
            dimension_semantics=("parallel",),
            flags={"XLA_TPU_STORE_TO_LOAD_FORWARDING_WINDOW": 12288},
        ),
    )(target_items_context, interacted_items_context, users3,
      mask01, wd, bd, W_mlp, bm)
    return out, attn


# speedup vs baseline: 1.1259x; 1.1054x over previous
import jax
import jax.numpy as jnp
from jax.experimental import pallas as pl
from jax.experimental.pallas import tpu as pltpu

_MASK_VALUE = -10000000.0


def _pooler_kernel(t_ref, k_ref, u_ref, mb_ref, wd_ref, bd_ref, wm_ref,
                   bm_ref, out_ref, attn_ref):
    BU = t_ref.shape[0]
    C = t_ref.shape[2]
    E = u_ref.shape[2]
    w1 = wd_ref[:, :C]
    w2 = wd_ref[:, C:]
    b = bd_ref[0, 0]
    for u in range(BU):
        t = t_ref[u]
        k = k_ref[u]
        s_t = jax.lax.dot_general(t, w1, (((1,), (1,)), ((), ())),
                                  preferred_element_type=jnp.float32)
        s_kb = jax.lax.dot_general(w2, k, (((1,), (1,)), ((), ())),
                                   preferred_element_type=jnp.float32) + b
        e = jnp.exp(jnp.tanh(s_t + s_kb)) * mb_ref[u]
        eb = e.astype(jnp.bfloat16)
        kb = k.astype(jnp.bfloat16)
        ones_j = jnp.ones((e.shape[1], 1), jnp.bfloat16)
        s = jnp.dot(eb, ones_j, preferred_element_type=jnp.float32)
        pooled_u = jnp.dot(eb, kb, preferred_element_type=jnp.float32)
        r = 1.0 / s
        attn_ref[u] = e * r
        u_part = jnp.dot(u_ref[u], wm_ref[:E, :],
                         preferred_element_type=jnp.float32)
        h = jnp.dot(pooled_u * r, wm_ref[E:, :],
                    preferred_element_type=jnp.float32)
        out_ref[u] = jnp.maximum(h + u_part + bm_ref[:], 0.0)


def kernel(target_items_context, interacted_items_context, user_embeds,
           attention_mask, w_dense, b_dense, W_mlp, b_mlp):
    U, I, C = target_items_context.shape
    J = interacted_items_context.shape[1]
    E = user_embeds.shape[1]
    BU = 8
    mask01 = attention_mask.astype(jnp.float32).reshape(U, 1, J)
    users3 = user_embeds.reshape(U, 1, E)
    wd = w_dense.reshape(1, 2 * C)
    bd = b_dense.reshape(1, 1)
    bm = b_mlp.reshape(1, C)
    out, attn = pl.pallas_call(
        _pooler_kernel,
        grid=(U // BU,),
        in_specs=[
            pl.BlockSpec((BU, I, C), lambda u: (u, 0, 0)),
            pl.BlockSpec((BU, J, C), lambda u: (u, 0, 0)),
            pl.BlockSpec((BU, 1, E), lambda u: (u, 0, 0)),
            pl.BlockSpec((BU, 1, J), lambda u: (u, 0, 0)),
            pl.BlockSpec((1, 2 * C), lambda u: (0, 0)),
            pl.BlockSpec((1, 1), lambda u: (0, 0)),
            pl.BlockSpec((E + C, C), lambda u: (0, 0)),
            pl.BlockSpec((1, C), lambda u: (0, 0)),
        ],
        out_specs=[
            pl.BlockSpec((BU, I, C), lambda u: (u, 0, 0)),
            pl.BlockSpec((BU, I, J), lambda u: (u, 0, 0)),
        ],
        out_shape=[
            jax.ShapeDtypeStruct((U, I, C), jnp.float32),
            jax.ShapeDtypeStruct((U, I, J), jnp.float32),
        ],
        compiler_params=pltpu.CompilerParams(
            dimension_semantics=("parallel",),
        ),
    )(target_items_context, interacted_items_context, users3,
      mask01, wd, bd, W_mlp, bm)
    return out, attn
